# baseline (speedup 1.0000x reference)
import functools

import jax
import jax.numpy as jnp
from jax import lax
from jax.experimental import pallas as pl
from jax.experimental.pallas import tpu as pltpu
from jax.experimental.pallas import tpu_sc as plsc

B = 64
L = 8192
HIST = 50
FUSE_WEIGHT = 0.5

RPW = 8
NWORK = B // RPW

NK = 64
KB = L // NK


def _sc_gather1(attMap_e, meta):
    mesh = plsc.VectorSubcoreMesh(core_axis_name="c", subcore_axis_name="s")

    @functools.partial(
        pl.kernel,
        out_type=jax.ShapeDtypeStruct((B, L), jnp.float32),
        mesh=mesh,
        scratch_types=[
            pltpu.VMEM((16,), jnp.int32),
            pltpu.VMEM((16,), jnp.int32),
            pltpu.VMEM((16,), jnp.int32),
            pltpu.VMEM((16,), jnp.int32),
            pltpu.VMEM((RPW, L), jnp.float32),
            pltpu.SemaphoreType.DMA,
        ],
    )
    def gather_kernel(att_hbm, meta_hbm, out_hbm,
                      tl_v, fpos_v, ids_v, idx_v, rows_v, sem):
        wid = lax.axis_index("s") * 2 + lax.axis_index("c")

        @pl.when(wid < NWORK)
        def _():
            c = wid // 2
            h = (wid % 2) * RPW
            pltpu.sync_copy(meta_hbm.at[pl.ds(c * 16, 16)], tl_v)
            tl = tl_v[...]
            bvec = c * 16 + lax.iota(jnp.int32, 16)
            fpos_v[...] = B + (bvec * HIST + tl - 1) * 2 + 1
            pltpu.async_copy(meta_hbm.at[fpos_v], ids_v, sem).wait()
            idx_v[...] = ids_v[...] - 1
            pltpu.async_copy(att_hbm.at[idx_v.at[pl.ds(h, RPW)]], rows_v,
                             sem).wait()
            pltpu.sync_copy(rows_v, out_hbm.at[pl.ds(wid * RPW, RPW)])

    return gather_kernel(attMap_e, meta)


def _mm_body(meta_ref, att_ref, att_any, out_ref, mask_ref, prob_v, sem_p):
    k = pl.program_id(0)

    @pl.when(k == 0)
    def _():
        rowio = lax.broadcasted_iota(jnp.int32, (B, 1), 0)

        def start(j, mvec):
            tl = meta_ref[j]
            pos2 = tl - 2 + HIST * jnp.maximum(2 - tl, 0)
            i2 = meta_ref[B + (j * HIST + pos2) * 2 + 1] - 1
            pltpu.make_async_copy(att_any.at[pl.ds(i2, 1), :],
                                  prob_v.at[pl.ds(j, 1), :], sem_p).start()
            return jnp.where(rowio == j, tl.astype(jnp.float32), mvec)

        tlvec = lax.fori_loop(0, B, start, jnp.zeros((B, 1), jnp.float32))
        mask_ref[...] = (tlvec >= 2.0).astype(jnp.float32)
        pltpu.make_async_copy(att_any.at[pl.ds(0, B), :], prob_v, sem_p).wait()
        out_ref[...] = jnp.zeros_like(out_ref)

    a = prob_v[:, pl.ds(k * KB, KB)].astype(jnp.bfloat16)
    bm = att_ref[...].astype(jnp.bfloat16)
    out_ref[...] += jnp.dot(a, bm, preferred_element_type=jnp.float32)


def _tc_matmul(meta, attMap_e):
    return pl.pallas_call(
        _mm_body,
        grid=(NK,),
        in_specs=[
            pl.BlockSpec(memory_space=pltpu.SMEM),
            pl.BlockSpec((KB, L), lambda k: (k, 0)),
            pl.BlockSpec(memory_space=pl.ANY),
        ],
        out_specs=[
            pl.BlockSpec((B, L), lambda k: (0, 0)),
            pl.BlockSpec((B, 1), lambda k: (0, 0)),
        ],
        out_shape=[
            jax.ShapeDtypeStruct((B, L), jnp.float32),
            jax.ShapeDtypeStruct((B, 1), jnp.float32),
        ],
        scratch_shapes=[
            pltpu.VMEM((B, L), jnp.float32),
            pltpu.SemaphoreType.DMA,
        ],
        compiler_params=pltpu.CompilerParams(
            vmem_limit_bytes=100 * 1024 * 1024,
        ),
    )(meta, attMap_e, attMap_e)


def _combine_body(mask_ref, p1_ref, acc_ref, adj_ref, out_ref):
    acc = acc_ref[...]
    mn2 = jnp.min(acc, axis=-1, keepdims=True)
    mx2 = jnp.max(acc, axis=-1, keepdims=True)
    p2 = (acc - mn2) / (mx2 - mn2)
    p1 = p1_ref[...]
    mn1 = jnp.min(p1, axis=-1, keepdims=True)
    mx1 = jnp.max(p1, axis=-1, keepdims=True)
    y1 = (p1 - mn1) / (mx1 - mn1)
    adj = adj_ref[...]
    wn = (adj - jnp.min(adj)) / (jnp.max(adj) - jnp.min(adj))
    mask = mask_ref[...]
    out_ref[...] = y1 + mask * (FUSE_WEIGHT * wn) * p2


def _tc_combine(maskcol, prob1, acc, adjust2):
    return pl.pallas_call(
        _combine_body,
        out_shape=jax.ShapeDtypeStruct((B, L), jnp.float32),
    )(maskcol, prob1, acc, adjust2)


def kernel(Final_output, attMap_e, adjust2, traj, traj_len):
    del Final_output
    meta = jnp.concatenate([traj_len.astype(jnp.int32),
                            traj.astype(jnp.int32).reshape(B * HIST * 2)])
    prob1 = _sc_gather1(attMap_e, meta)
    acc, maskcol = _tc_matmul(meta, attMap_e)
    return _tc_combine(maskcol, prob1, acc, adjust2)


# speedup vs baseline: 1.1054x; 1.1054x over previous
import functools

import jax
import jax.numpy as jnp
from jax import lax
from jax.experimental import pallas as pl
from jax.experimental.pallas import tpu as pltpu
from jax.experimental.pallas import tpu_sc as plsc

B = 64
L = 8192
HIST = 50
FUSE_WEIGHT = 0.5

RPW = 8
NWORK = B // RPW

NK = 32
KB = L // NK


def _sc_gather1(attMap_e, meta):
    mesh = plsc.VectorSubcoreMesh(core_axis_name="c", subcore_axis_name="s")

    @functools.partial(
        pl.kernel,
        out_type=jax.ShapeDtypeStruct((B, L), jnp.float32),
        mesh=mesh,
        scratch_types=[
            pltpu.VMEM((16,), jnp.int32),
            pltpu.VMEM((16,), jnp.int32),
            pltpu.VMEM((16,), jnp.int32),
            pltpu.VMEM((16,), jnp.int32),
            pltpu.VMEM((RPW, L), jnp.float32),
            pltpu.SemaphoreType.DMA,
        ],
    )
    def gather_kernel(att_hbm, meta_hbm, out_hbm,
                      tl_v, fpos_v, ids_v, idx_v, rows_v, sem):
        wid = lax.axis_index("s") * 2 + lax.axis_index("c")

        @pl.when(wid < NWORK)
        def _():
            c = wid // 2
            h = (wid % 2) * RPW
            pltpu.sync_copy(meta_hbm.at[pl.ds(c * 16, 16)], tl_v)
            tl = tl_v[...]
            bvec = c * 16 + lax.iota(jnp.int32, 16)
            fpos_v[...] = B + (bvec * HIST + tl - 1) * 2 + 1
            pltpu.async_copy(meta_hbm.at[fpos_v], ids_v, sem).wait()
            idx_v[...] = ids_v[...] - 1
            pltpu.async_copy(att_hbm.at[idx_v.at[pl.ds(h, RPW)]], rows_v,
                             sem).wait()
            pltpu.sync_copy(rows_v, out_hbm.at[pl.ds(wid * RPW, RPW)])

    return gather_kernel(attMap_e, meta)


def _mm_body(meta_ref, att_ref, att_any, adj_ref, out_ref, mask_v, prob_v,
             sem_p):
    k = pl.program_id(0)

    @pl.when(k == 0)
    def _():
        rowio = lax.broadcasted_iota(jnp.int32, (B, 1), 0)

        def start(j, mvec):
            tl = meta_ref[j]
            pos2 = tl - 2 + HIST * jnp.maximum(2 - tl, 0)
            i2 = meta_ref[B + (j * HIST + pos2) * 2 + 1] - 1
            pltpu.make_async_copy(att_any.at[pl.ds(i2, 1), :],
                                  prob_v.at[pl.ds(j, 1), :], sem_p).start()
            return jnp.where(rowio == j, tl.astype(jnp.float32), mvec)

        tlvec = lax.fori_loop(0, B, start, jnp.zeros((B, 1), jnp.float32))
        mask_v[...] = (tlvec >= 2.0).astype(jnp.float32)
        pltpu.make_async_copy(att_any.at[pl.ds(0, B), :], prob_v, sem_p).wait()
        out_ref[...] = jnp.zeros_like(out_ref)

    a = prob_v[:, pl.ds(k * KB, KB)].astype(jnp.bfloat16)
    bm = att_ref[...].astype(jnp.bfloat16)
    out_ref[...] += jnp.dot(a, bm, preferred_element_type=jnp.float32)

    @pl.when(k == NK - 1)
    def _():
        acc = out_ref[...]
        mn2 = jnp.min(acc, axis=-1, keepdims=True)
        mx2 = jnp.max(acc, axis=-1, keepdims=True)
        p2 = (acc - mn2) / (mx2 - mn2)
        adj = adj_ref[...]
        wn = (adj - jnp.min(adj)) / (jnp.max(adj) - jnp.min(adj))
        out_ref[...] = mask_v[...] * (FUSE_WEIGHT * wn) * p2


def _tc_matmul(meta, attMap_e, adjust2):
    return pl.pallas_call(
        _mm_body,
        grid=(NK,),
        in_specs=[
            pl.BlockSpec(memory_space=pltpu.SMEM),
            pl.BlockSpec((KB, L), lambda k: (k, 0)),
            pl.BlockSpec(memory_space=pl.ANY),
            pl.BlockSpec((1, L), lambda k: (0, 0)),
        ],
        out_specs=pl.BlockSpec((B, L), lambda k: (0, 0)),
        out_shape=jax.ShapeDtypeStruct((B, L), jnp.float32),
        scratch_shapes=[
            pltpu.VMEM((B, 1), jnp.float32),
            pltpu.VMEM((B, L), jnp.float32),
            pltpu.SemaphoreType.DMA,
        ],
        compiler_params=pltpu.CompilerParams(
            vmem_limit_bytes=100 * 1024 * 1024,
        ),
    )(meta, attMap_e, attMap_e, adjust2)


def _combine_body(p1_ref, z_ref, out_ref):
    p1 = p1_ref[...]
    mn1 = jnp.min(p1, axis=-1, keepdims=True)
    mx1 = jnp.max(p1, axis=-1, keepdims=True)
    out_ref[...] = (p1 - mn1) / (mx1 - mn1) + z_ref[...]


def _tc_combine(prob1, z):
    return pl.pallas_call(
        _combine_body,
        out_shape=jax.ShapeDtypeStruct((B, L), jnp.float32),
    )(prob1, z)


def kernel(Final_output, attMap_e, adjust2, traj, traj_len):
    del Final_output
    meta = jnp.concatenate([traj_len.astype(jnp.int32),
                            traj.astype(jnp.int32).reshape(B * HIST * 2)])
    prob1 = _sc_gather1(attMap_e, meta)
    z = _tc_matmul(meta, attMap_e, adjust2)
    return _tc_combine(prob1, z)


# speedup vs baseline: 1.1137x; 1.0076x over previous
import functools

import jax
import jax.numpy as jnp
from jax import lax
from jax.experimental import pallas as pl
from jax.experimental.pallas import tpu as pltpu
from jax.experimental.pallas import tpu_sc as plsc

B = 64
L = 8192
HIST = 50
FUSE_WEIGHT = 0.5

RPW = 8
NWORK = B // RPW

NK = 32
KB = L // NK


def _sc_gather1(attMap_e, meta):
    mesh = plsc.VectorSubcoreMesh(core_axis_name="c", subcore_axis_name="s")

    @functools.partial(
        pl.kernel,
        out_type=jax.ShapeDtypeStruct((B, L), jnp.float32),
        mesh=mesh,
        scratch_types=[
            pltpu.VMEM((16,), jnp.int32),
            pltpu.VMEM((16,), jnp.int32),
            pltpu.VMEM((16,), jnp.int32),
            pltpu.VMEM((16,), jnp.int32),
            pltpu.VMEM((RPW, L), jnp.float32),
            pltpu.SemaphoreType.DMA,
        ],
    )
    def gather_kernel(att_hbm, meta_hbm, out_hbm,
                      tl_v, fpos_v, ids_v, idx_v, rows_v, sem):
        wid = lax.axis_index("s") * 2 + lax.axis_index("c")

        @pl.when(wid < NWORK)
        def _():
            c = wid // 2
            h = (wid % 2) * RPW
            pltpu.sync_copy(meta_hbm.at[pl.ds(c * 16, 16)], tl_v)
            tl = tl_v[...]
            bvec = c * 16 + lax.iota(jnp.int32, 16)
            fpos_v[...] = B + (bvec * HIST + tl - 1) * 2 + 1
            pltpu.async_copy(meta_hbm.at[fpos_v], ids_v, sem).wait()
            idx_v[...] = ids_v[...] - 1
            pltpu.async_copy(att_hbm.at[idx_v.at[pl.ds(h, RPW)]], rows_v,
                             sem).wait()
            pltpu.sync_copy(rows_v, out_hbm.at[pl.ds(wid * RPW, RPW)])

    return gather_kernel(attMap_e, meta)


def _mm_body(meta_ref, att_ref, att_any, adj_ref, out_ref, mask_v, prob_v,
             sem_p):
    k = pl.program_id(0)

    @pl.when(k == 0)
    def _():
        rowio = lax.broadcasted_iota(jnp.int32, (B, 1), 0)

        def start(j, mvec):
            tl = meta_ref[j]
            pos2 = tl - 2 + HIST * jnp.maximum(2 - tl, 0)
            i2 = meta_ref[B + (j * HIST + pos2) * 2 + 1] - 1
            pltpu.make_async_copy(att_any.at[pl.ds(i2, 1), :],
                                  prob_v.at[pl.ds(j, 1), :], sem_p).start()
            return jnp.where(rowio == j, tl.astype(jnp.float32), mvec)

        tlvec = lax.fori_loop(0, B, start, jnp.zeros((B, 1), jnp.float32))
        mask_v[...] = (tlvec >= 2.0).astype(jnp.float32)
        pltpu.make_async_copy(att_any.at[pl.ds(0, B), :], prob_v, sem_p).wait()
        out_ref[...] = jnp.zeros_like(out_ref)

    a = prob_v[:, pl.ds(k * KB, KB)].astype(jnp.bfloat16)
    bm = att_ref[...].astype(jnp.bfloat16)
    out_ref[...] += jnp.dot(a, bm, preferred_element_type=jnp.float32)

    @pl.when(k == NK - 1)
    def _():
        acc = out_ref[...]
        mn2 = jnp.min(acc, axis=-1, keepdims=True)
        mx2 = jnp.max(acc, axis=-1, keepdims=True)
        p2 = (acc - mn2) / (mx2 - mn2)
        adj = adj_ref[...]
        wn = (adj - jnp.min(adj)) / (jnp.max(adj) - jnp.min(adj))
        out_ref[...] = mask_v[...] * (FUSE_WEIGHT * wn) * p2


def _tc_matmul(meta, attMap_e, adjust2):
    return pl.pallas_call(
        _mm_body,
        grid=(NK,),
        in_specs=[
            pl.BlockSpec(memory_space=pltpu.SMEM),
            pl.BlockSpec((KB, L), lambda k: (k, 0)),
            pl.BlockSpec(memory_space=pl.ANY),
            pl.BlockSpec((1, L), lambda k: (0, 0)),
        ],
        out_specs=pl.BlockSpec((B, L), lambda k: (0, 0)),
        out_shape=jax.ShapeDtypeStruct((B, L), jnp.float32),
        scratch_shapes=[
            pltpu.VMEM((B, 1), jnp.float32),
            pltpu.VMEM((B, L), jnp.float32),
            pltpu.SemaphoreType.DMA,
        ],
        compiler_params=pltpu.CompilerParams(
            vmem_limit_bytes=100 * 1024 * 1024,
        ),
    )(meta, attMap_e, attMap_e, adjust2)


def _combine_body(p1_ref, z_ref, out_ref):
    p1 = p1_ref[...]
    mn1 = jnp.min(p1, axis=-1, keepdims=True)
    mx1 = jnp.max(p1, axis=-1, keepdims=True)
    out_ref[...] = (p1 - mn1) / (mx1 - mn1) + z_ref[...]


def _tc_combine(prob1, z):
    return pl.pallas_call(
        _combine_body,
        out_shape=jax.ShapeDtypeStruct((B, L), jnp.float32),
    )(prob1, z)


def kernel(Final_output, attMap_e, adjust2, traj, traj_len):
    del Final_output
    meta = jnp.concatenate([traj_len.astype(jnp.int32),
                            traj.astype(jnp.int32).reshape(B * HIST * 2)])
    z = _tc_matmul(meta, attMap_e, adjust2)
    prob1 = _sc_gather1(attMap_e, meta)
    return _tc_combine(prob1, z)
